# baseline (speedup 1.0000x reference)
import functools
import numpy as np
import jax
import jax.numpy as jnp
from jax import lax
from jax.experimental import pallas as pl
from jax.experimental.pallas import tpu as pltpu
from jax.experimental.pallas import tpu_sc as plsc

B, N, K, D = 16, 512, 32, 128
C = 128
NCHUNK = (B * N) // C
NW = 32
CHUNKS_PER_W = NCHUNK // NW
PAIRS = C * K
LANES = 16
GROUPS = PAIRS // LANES
RBLK = C // LANES


def _sc_scatter_body(cols_hbm, vals_hbm, m_hbm,
                     idx_v, val_v, acc_v):
    wid = lax.axis_index("s") * 2 + lax.axis_index("c")
    for cc in range(CHUNKS_PER_W):
        chunk = wid * CHUNKS_PER_W + cc
        base = chunk * PAIRS
        pltpu.sync_copy(cols_hbm.at[pl.ds(base, PAIRS)], idx_v)
        pltpu.sync_copy(vals_hbm.at[pl.ds(base, PAIRS)], val_v)

        zeros = jnp.zeros((LANES,), jnp.float32)

        def zero_blk(i, carry):
            base0 = i * (LANES * 16)
            for j in range(16):
                acc_v[pl.ds(base0 + j * LANES, LANES)] = zeros
            return carry

        lax.fori_loop(0, (C * N) // (LANES * 16), zero_blk, 0)

        def group(g, carry):
            off = g * LANES
            cols = idx_v[pl.ds(off, LANES)]
            vals = val_v[pl.ds(off, LANES)]
            rowbase = (g // (K // LANES)) * N
            plsc.addupdate_scatter(acc_v, [rowbase + cols], vals)
            return carry

        lax.fori_loop(0, GROUPS, group, 0)
        pltpu.sync_copy(acc_v, m_hbm.at[chunk])


def _build_m_sc(cols_flat, vals_flat):
    mesh = plsc.VectorSubcoreMesh(core_axis_name="c", subcore_axis_name="s",
                                  num_cores=2, num_subcores=16)
    k = pl.kernel(
        _sc_scatter_body,
        out_type=jax.ShapeDtypeStruct((NCHUNK, C * N), jnp.float32),
        mesh=mesh,
        scratch_types=[
            pltpu.VMEM((PAIRS,), jnp.int32),
            pltpu.VMEM((PAIRS,), jnp.float32),
            pltpu.VMEM((C * N,), jnp.float32),
        ],
        compiler_params=pltpu.CompilerParams(
            needs_layout_passes=False, use_tc_tiling_on_sc=False),
    )
    return k(cols_flat, vals_flat)


def _mm_body(m_ref, h_ref, o_ref):
    m = m_ref[0].astype(jnp.bfloat16)
    h1 = jnp.dot(m, h_ref[0].astype(jnp.bfloat16),
                 preferred_element_type=jnp.float32)
    o_ref[0] = jnp.dot(m, h1.astype(jnp.bfloat16),
                       preferred_element_type=jnp.float32)


def _two_layer_mm(m, h):
    return pl.pallas_call(
        _mm_body,
        grid=(B,),
        in_specs=[
            pl.BlockSpec((1, N, N), lambda b: (b, 0, 0)),
            pl.BlockSpec((1, N, D), lambda b: (b, 0, 0)),
        ],
        out_specs=pl.BlockSpec((1, N, D), lambda b: (b, 0, 0)),
        out_shape=jax.ShapeDtypeStruct((B, N, D), jnp.float32),
    )(m, h)


@jax.jit
def kernel(node_hidden, in_node_index, in_node_mask):
    cols_flat = in_node_index.reshape(-1)
    vals_flat = in_node_mask.reshape(-1)

    m = _build_m_sc(cols_flat, vals_flat)
    m = m.reshape(B, N, N)
    return _two_layer_mm(m, node_hidden)


# speedup vs baseline: 1.0006x; 1.0006x over previous
import functools
import numpy as np
import jax
import jax.numpy as jnp
from jax import lax
from jax.experimental import pallas as pl
from jax.experimental.pallas import tpu as pltpu
from jax.experimental.pallas import tpu_sc as plsc

B, N, K, D = 16, 512, 32, 128
C = 128
NCHUNK = (B * N) // C
NW = 32
CHUNKS_PER_W = NCHUNK // NW
PAIRS = C * K
LANES = 16
GROUPS = PAIRS // LANES
RBLK = C // LANES


def _sc_scatter_body(cols_hbm, vals_hbm, m_hbm,
                     idx_v, val_v, acc_v):
    wid = lax.axis_index("s") * 2 + lax.axis_index("c")
    for cc in range(CHUNKS_PER_W):
        chunk = wid * CHUNKS_PER_W + cc
        base = chunk * PAIRS
        pltpu.sync_copy(cols_hbm.at[pl.ds(base, PAIRS)], idx_v)
        pltpu.sync_copy(vals_hbm.at[pl.ds(base, PAIRS)], val_v)

        zeros = jnp.zeros((LANES,), jnp.float32)

        def zero_blk(i, carry):
            base0 = i * (LANES * 16)
            for j in range(16):
                acc_v[pl.ds(base0 + j * LANES, LANES)] = zeros
            return carry

        lax.fori_loop(0, (C * N) // (LANES * 16), zero_blk, 0)

        def group(g, carry):
            off = g * LANES
            cols = idx_v[pl.ds(off, LANES)]
            vals = val_v[pl.ds(off, LANES)]
            rowbase = (g // (K // LANES)) * N
            plsc.addupdate_scatter(acc_v, [rowbase + cols], vals)
            return carry

        lax.fori_loop(0, GROUPS, group, 0)
        pltpu.sync_copy(acc_v, m_hbm.at[chunk])


def _build_m_sc(cols_flat, vals_flat):
    mesh = plsc.VectorSubcoreMesh(core_axis_name="c", subcore_axis_name="s",
                                  num_cores=2, num_subcores=16)
    k = pl.kernel(
        _sc_scatter_body,
        out_type=jax.ShapeDtypeStruct((NCHUNK, C * N), jnp.float32),
        mesh=mesh,
        scratch_types=[
            pltpu.VMEM((PAIRS,), jnp.int32),
            pltpu.VMEM((PAIRS,), jnp.float32),
            pltpu.VMEM((C * N,), jnp.float32),
        ],
        compiler_params=pltpu.CompilerParams(
            needs_layout_passes=False, use_tc_tiling_on_sc=False),
    )
    return k(cols_flat, vals_flat)


def _mm_body(m_ref, h_ref, o_ref):
    m = m_ref[0]
    h1 = jnp.dot(m, h_ref[0], preferred_element_type=jnp.float32)
    o_ref[0] = jnp.dot(m, h1, preferred_element_type=jnp.float32)


def _two_layer_mm(m, h):
    return pl.pallas_call(
        _mm_body,
        grid=(B,),
        in_specs=[
            pl.BlockSpec((1, N, N), lambda b: (b, 0, 0)),
            pl.BlockSpec((1, N, D), lambda b: (b, 0, 0)),
        ],
        out_specs=pl.BlockSpec((1, N, D), lambda b: (b, 0, 0)),
        out_shape=jax.ShapeDtypeStruct((B, N, D), jnp.float32),
    )(m, h)


@jax.jit
def kernel(node_hidden, in_node_index, in_node_mask):
    cols_flat = in_node_index.reshape(-1)
    vals_flat = in_node_mask.reshape(-1)

    m = _build_m_sc(cols_flat, vals_flat)
    m = m.reshape(B, N, N)
    return _two_layer_mm(m, node_hidden)


# speedup vs baseline: 1.0153x; 1.0147x over previous
import functools
import numpy as np
import jax
import jax.numpy as jnp
from jax import lax
from jax.experimental import pallas as pl
from jax.experimental.pallas import tpu as pltpu
from jax.experimental.pallas import tpu_sc as plsc

B, N, K, D = 16, 512, 32, 128
C = 64
NCHUNK = (B * N) // C
NW = 32
CHUNKS_PER_W = NCHUNK // NW
PAIRS = C * K
LANES = 16
GROUPS = PAIRS // LANES


def _sc_scatter_body(cols_hbm, vals_hbm, m_hbm,
                     idx_v, val_v, acc_v, sems):
    wid = lax.axis_index("s") * 2 + lax.axis_index("c")
    zeros = jnp.zeros((LANES,), jnp.float32)

    for cc in range(CHUNKS_PER_W):
        buf = cc % 2
        chunk = wid * CHUNKS_PER_W + cc
        base = chunk * PAIRS

        if cc >= 2:
            prev = chunk - 2
            pltpu.make_async_copy(
                acc_v.at[buf], m_hbm.at[prev], sems.at[buf]).wait()
            def unzero(g, carry):
                off = g * LANES
                cols = idx_v[buf, pl.ds(off, LANES)]
                rowbase = (g // (K // LANES)) * N
                plsc.store_scatter(acc_v.at[buf], [rowbase + cols], zeros)
                return carry
            lax.fori_loop(0, GROUPS, unzero, 0)
        else:
            def zero_blk(i, carry):
                base0 = i * (LANES * 16)
                for j in range(16):
                    acc_v[buf, pl.ds(base0 + j * LANES, LANES)] = zeros
                return carry
            lax.fori_loop(0, (C * N) // (LANES * 16), zero_blk, 0)

        pltpu.sync_copy(cols_hbm.at[pl.ds(base, PAIRS)], idx_v.at[buf])
        pltpu.sync_copy(vals_hbm.at[pl.ds(base, PAIRS)], val_v.at[buf])

        def group(g, carry):
            off = g * LANES
            cols = idx_v[buf, pl.ds(off, LANES)]
            vals = val_v[buf, pl.ds(off, LANES)]
            rowbase = (g // (K // LANES)) * N
            plsc.addupdate_scatter(acc_v.at[buf], [rowbase + cols], vals)
            return carry

        lax.fori_loop(0, GROUPS, group, 0)
        pltpu.async_copy(acc_v.at[buf], m_hbm.at[chunk], sems.at[buf])

    for cc in range(CHUNKS_PER_W - 2, CHUNKS_PER_W):
        buf = cc % 2
        chunk = wid * CHUNKS_PER_W + cc
        pltpu.make_async_copy(
            acc_v.at[buf], m_hbm.at[chunk], sems.at[buf]).wait()


def _build_m_sc(cols_flat, vals_flat):
    mesh = plsc.VectorSubcoreMesh(core_axis_name="c", subcore_axis_name="s",
                                  num_cores=2, num_subcores=16)
    k = pl.kernel(
        _sc_scatter_body,
        out_type=jax.ShapeDtypeStruct((NCHUNK, C * N), jnp.float32),
        mesh=mesh,
        scratch_types=[
            pltpu.VMEM((2, PAIRS), jnp.int32),
            pltpu.VMEM((2, PAIRS), jnp.float32),
            pltpu.VMEM((2, C * N), jnp.float32),
            pltpu.SemaphoreType.DMA((2,)),
        ],
        compiler_params=pltpu.CompilerParams(
            needs_layout_passes=False, use_tc_tiling_on_sc=False),
    )
    return k(cols_flat, vals_flat)


def _mm_body(m_ref, h_ref, o_ref):
    m = m_ref[0]
    h1 = jnp.dot(m, h_ref[0], preferred_element_type=jnp.float32)
    o_ref[0] = jnp.dot(m, h1, preferred_element_type=jnp.float32)


def _two_layer_mm(m, h):
    return pl.pallas_call(
        _mm_body,
        grid=(B,),
        in_specs=[
            pl.BlockSpec((1, N, N), lambda b: (b, 0, 0)),
            pl.BlockSpec((1, N, D), lambda b: (b, 0, 0)),
        ],
        out_specs=pl.BlockSpec((1, N, D), lambda b: (b, 0, 0)),
        out_shape=jax.ShapeDtypeStruct((B, N, D), jnp.float32),
    )(m, h)


@jax.jit
def kernel(node_hidden, in_node_index, in_node_mask):
    cols_flat = in_node_index.reshape(-1)
    vals_flat = in_node_mask.reshape(-1)

    m = _build_m_sc(cols_flat, vals_flat)
    m = m.reshape(B, N, N)
    return _two_layer_mm(m, node_hidden)
